# baseline (speedup 1.0000x reference)
import jax
import jax.numpy as jnp
from jax.experimental import pallas as pl
from jax.experimental.pallas import tpu as pltpu

_B = 4096
_EMB = 128
_ND = 5000
_NA = 8000
_B_TILE = 512
_KD = 640
_KA = 1024
_NK = 8
_NB = _B // _B_TILE
_NDP = _KD * _NK
_NAP = _KA * _NK


def _fused(rate_ref, d_ref, a_ref, er_ref, wd_ref, wa_ref, od_ref, oa_ref,
           out_ref, acc_d, acc_a, rs_d, rs_a):
    k = pl.program_id(1)

    d = d_ref[...].astype(jnp.bfloat16)
    a = a_ref[...].astype(jnp.bfloat16)

    dn_t = (((1,), (1,)), ((), ()))
    dn_n = (((1,), (0,)), ((), ()))
    pd = jax.lax.dot_general(d, wd_ref[...], dn_t, preferred_element_type=jnp.float32)
    pa = jax.lax.dot_general(a, wa_ref[...], dn_t, preferred_element_type=jnp.float32)
    sd = jax.lax.dot_general(d, od_ref[...], dn_n, preferred_element_type=jnp.float32)
    sa = jax.lax.dot_general(a, oa_ref[...], dn_n, preferred_element_type=jnp.float32)

    @pl.when(k == 0)
    def _init():
        acc_d[...] = pd
        acc_a[...] = pa
        rs_d[...] = sd
        rs_a[...] = sa
        classes = jax.lax.broadcasted_iota(jnp.int32, (1, 8), 1)
        onehot = (rate_ref[...] == classes).astype(jnp.float32)
        out_ref[:, 0:_EMB] = jax.lax.dot_general(
            onehot, er_ref[...], dn_n, preferred_element_type=jnp.float32)

    @pl.when(k != 0)
    def _acc():
        acc_d[...] += pd
        acc_a[...] += pa
        rs_d[...] += sd
        rs_a[...] += sa

    @pl.when(k == _NK - 1)
    def _fin():
        out_ref[:, _EMB:2 * _EMB] = acc_d[...] / rs_d[:, 0:1]
        out_ref[:, 2 * _EMB:3 * _EMB] = acc_a[...] / rs_a[:, 0:1]


def kernel(rate_idx, director_idx, actors_idx, emb_rate, W_director, W_actor):
    rate2d = rate_idx.astype(jnp.int32).reshape(_B, 1)
    er_pad = jnp.pad(emb_rate, ((0, 2), (0, 0)))
    wd_pad = jnp.pad(W_director.astype(jnp.bfloat16), ((0, 0), (0, _NDP - _ND)))
    wa_pad = jnp.pad(W_actor.astype(jnp.bfloat16), ((0, 0), (0, _NAP - _NA)))
    ones_d = jnp.pad(jnp.ones((_ND, 8), jnp.bfloat16), ((0, _NDP - _ND), (0, 0)))
    ones_a = jnp.pad(jnp.ones((_NA, 8), jnp.bfloat16), ((0, _NAP - _NA), (0, 0)))

    return pl.pallas_call(
        _fused,
        grid=(_NB, _NK),
        in_specs=[
            pl.BlockSpec((_B_TILE, 1), lambda i, k: (i, 0)),
            pl.BlockSpec((_B_TILE, _KD), lambda i, k: (i, k)),
            pl.BlockSpec((_B_TILE, _KA), lambda i, k: (i, k)),
            pl.BlockSpec((8, _EMB), lambda i, k: (0, 0)),
            pl.BlockSpec((_EMB, _KD), lambda i, k: (0, k)),
            pl.BlockSpec((_EMB, _KA), lambda i, k: (0, k)),
            pl.BlockSpec((_KD, 8), lambda i, k: (k, 0)),
            pl.BlockSpec((_KA, 8), lambda i, k: (k, 0)),
        ],
        out_specs=pl.BlockSpec((_B_TILE, 3 * _EMB), lambda i, k: (i, 0)),
        out_shape=jax.ShapeDtypeStruct((_B, 3 * _EMB), jnp.float32),
        scratch_shapes=[
            pltpu.VMEM((_B_TILE, _EMB), jnp.float32),
            pltpu.VMEM((_B_TILE, _EMB), jnp.float32),
            pltpu.VMEM((_B_TILE, 8), jnp.float32),
            pltpu.VMEM((_B_TILE, 8), jnp.float32),
        ],
        compiler_params=pltpu.CompilerParams(
            dimension_semantics=("parallel", "arbitrary")),
    )(rate2d, director_idx, actors_idx, er_pad, wd_pad, wa_pad, ones_d, ones_a)


# speedup vs baseline: 1.0745x; 1.0745x over previous
import jax
import jax.numpy as jnp
from jax.experimental import pallas as pl
from jax.experimental.pallas import tpu as pltpu

_B = 4096
_EMB = 128
_ND = 5000
_NA = 8000
_B_TILE = 512
_KD = 1280
_KA = 2048
_NK = 4
_NB = _B // _B_TILE
_NDP = _KD * _NK
_NAP = _KA * _NK


def _fused(rate_ref, d_ref, a_ref, er_ref, wd_ref, wa_ref, od_ref, oa_ref,
           out_ref, acc_d, acc_a, rs_d, rs_a):
    k = pl.program_id(1)

    d = d_ref[...].astype(jnp.bfloat16)
    a = a_ref[...].astype(jnp.bfloat16)

    dn_t = (((1,), (1,)), ((), ()))
    dn_n = (((1,), (0,)), ((), ()))
    pd = jax.lax.dot_general(d, wd_ref[...], dn_t, preferred_element_type=jnp.float32)
    pa = jax.lax.dot_general(a, wa_ref[...], dn_t, preferred_element_type=jnp.float32)
    sd = jax.lax.dot_general(d, od_ref[...], dn_n, preferred_element_type=jnp.float32)
    sa = jax.lax.dot_general(a, oa_ref[...], dn_n, preferred_element_type=jnp.float32)

    @pl.when(k == 0)
    def _init():
        acc_d[...] = pd
        acc_a[...] = pa
        rs_d[...] = sd
        rs_a[...] = sa
        classes = jax.lax.broadcasted_iota(jnp.int32, (1, 8), 1)
        onehot = (rate_ref[...] == classes).astype(jnp.float32)
        out_ref[:, 0:_EMB] = jax.lax.dot_general(
            onehot, er_ref[...], dn_n, preferred_element_type=jnp.float32)

    @pl.when(k != 0)
    def _acc():
        acc_d[...] += pd
        acc_a[...] += pa
        rs_d[...] += sd
        rs_a[...] += sa

    @pl.when(k == _NK - 1)
    def _fin():
        out_ref[:, _EMB:2 * _EMB] = acc_d[...] / rs_d[:, 0:1]
        out_ref[:, 2 * _EMB:3 * _EMB] = acc_a[...] / rs_a[:, 0:1]


def kernel(rate_idx, director_idx, actors_idx, emb_rate, W_director, W_actor):
    rate2d = rate_idx.astype(jnp.int32).reshape(_B, 1)
    er_pad = jnp.pad(emb_rate, ((0, 2), (0, 0)))
    wd_pad = jnp.pad(W_director.astype(jnp.bfloat16), ((0, 0), (0, _NDP - _ND)))
    wa_pad = jnp.pad(W_actor.astype(jnp.bfloat16), ((0, 0), (0, _NAP - _NA)))
    ones_d = jnp.pad(jnp.ones((_ND, 8), jnp.bfloat16), ((0, _NDP - _ND), (0, 0)))
    ones_a = jnp.pad(jnp.ones((_NA, 8), jnp.bfloat16), ((0, _NAP - _NA), (0, 0)))

    return pl.pallas_call(
        _fused,
        grid=(_NB, _NK),
        in_specs=[
            pl.BlockSpec((_B_TILE, 1), lambda i, k: (i, 0)),
            pl.BlockSpec((_B_TILE, _KD), lambda i, k: (i, k)),
            pl.BlockSpec((_B_TILE, _KA), lambda i, k: (i, k)),
            pl.BlockSpec((8, _EMB), lambda i, k: (0, 0)),
            pl.BlockSpec((_EMB, _KD), lambda i, k: (0, k)),
            pl.BlockSpec((_EMB, _KA), lambda i, k: (0, k)),
            pl.BlockSpec((_KD, 8), lambda i, k: (k, 0)),
            pl.BlockSpec((_KA, 8), lambda i, k: (k, 0)),
        ],
        out_specs=pl.BlockSpec((_B_TILE, 3 * _EMB), lambda i, k: (i, 0)),
        out_shape=jax.ShapeDtypeStruct((_B, 3 * _EMB), jnp.float32),
        scratch_shapes=[
            pltpu.VMEM((_B_TILE, _EMB), jnp.float32),
            pltpu.VMEM((_B_TILE, _EMB), jnp.float32),
            pltpu.VMEM((_B_TILE, 8), jnp.float32),
            pltpu.VMEM((_B_TILE, 8), jnp.float32),
        ],
        compiler_params=pltpu.CompilerParams(
            dimension_semantics=("parallel", "arbitrary")),
    )(rate2d, director_idx, actors_idx, er_pad, wd_pad, wa_pad, ones_d, ones_a)


# speedup vs baseline: 1.1888x; 1.1064x over previous
import jax
import jax.numpy as jnp
from jax.experimental import pallas as pl
from jax.experimental.pallas import tpu as pltpu

_B = 4096
_EMB = 128
_ND = 5000
_NA = 8000
_B_TILE = 256
_NB = _B // _B_TILE


def _fused(rate_ref, d_ref, a_ref, er_ref, wd_ref, wa_ref, od_ref, oa_ref,
           out_ref):
    d = d_ref[...].astype(jnp.bfloat16)
    a = a_ref[...].astype(jnp.bfloat16)

    dn_t = (((1,), (1,)), ((), ()))
    dn_n = (((1,), (0,)), ((), ()))
    pd = jax.lax.dot_general(d, wd_ref[...], dn_t, preferred_element_type=jnp.float32)
    pa = jax.lax.dot_general(a, wa_ref[...], dn_t, preferred_element_type=jnp.float32)
    sd = jax.lax.dot_general(d, od_ref[...], dn_n, preferred_element_type=jnp.float32)
    sa = jax.lax.dot_general(a, oa_ref[...], dn_n, preferred_element_type=jnp.float32)

    classes = jax.lax.broadcasted_iota(jnp.int32, (1, 8), 1)
    onehot = (rate_ref[...] == classes).astype(jnp.float32)
    out_ref[:, 0:_EMB] = jax.lax.dot_general(
        onehot, er_ref[...], dn_n, preferred_element_type=jnp.float32)
    out_ref[:, _EMB:2 * _EMB] = pd / sd[:, 0:1]
    out_ref[:, 2 * _EMB:3 * _EMB] = pa / sa[:, 0:1]


def kernel(rate_idx, director_idx, actors_idx, emb_rate, W_director, W_actor):
    rate2d = rate_idx.astype(jnp.int32).reshape(_B, 1)
    er_pad = jnp.pad(emb_rate, ((0, 2), (0, 0)))
    wd = W_director.astype(jnp.bfloat16)
    wa = W_actor.astype(jnp.bfloat16)
    ones_d = jnp.ones((_ND, 8), jnp.bfloat16)
    ones_a = jnp.ones((_NA, 8), jnp.bfloat16)

    return pl.pallas_call(
        _fused,
        grid=(_NB,),
        in_specs=[
            pl.BlockSpec((_B_TILE, 1), lambda i: (i, 0)),
            pl.BlockSpec((_B_TILE, _ND), lambda i: (i, 0)),
            pl.BlockSpec((_B_TILE, _NA), lambda i: (i, 0)),
            pl.BlockSpec((8, _EMB), lambda i: (0, 0)),
            pl.BlockSpec((_EMB, _ND), lambda i: (0, 0)),
            pl.BlockSpec((_EMB, _NA), lambda i: (0, 0)),
            pl.BlockSpec((_ND, 8), lambda i: (0, 0)),
            pl.BlockSpec((_NA, 8), lambda i: (0, 0)),
        ],
        out_specs=pl.BlockSpec((_B_TILE, 3 * _EMB), lambda i: (i, 0)),
        out_shape=jax.ShapeDtypeStruct((_B, 3 * _EMB), jnp.float32),
        compiler_params=pltpu.CompilerParams(
            dimension_semantics=("parallel",)),
    )(rate2d, director_idx, actors_idx, er_pad, wd, wa, ones_d, ones_a)
